# baseline (speedup 1.0000x reference)
import functools

import jax
import jax.numpy as jnp
from jax import lax
from jax.experimental import pallas as pl
from jax.experimental.pallas import tpu as pltpu
from jax.experimental.pallas import tpu_sc as plsc

B, P, N, G = 16, 16, 16384, 64
G3 = G * G * G
LANES = 16
NW = 32
PAIRS_PER_W = (B * P) // NW
CHUNK = 2048
IDXROW = 128
NCOPY = CHUNK // IDXROW


def _floor_f32(x):
    t = x.astype(jnp.int32)
    tf = t.astype(jnp.float32)
    return jnp.where(tf > x, t - 1, t)


def _sc_body(pts_hbm, par_hbm, tab_hbm, out_hbm,
             pts_v, par_v, idx_v, rx_v, ry_v, rz_v, rows_v, acc_v, sem):
    cid = lax.axis_index("c")
    sid = lax.axis_index("s")
    wid = cid * 16 + sid
    b = wid // 2
    p0 = (wid % 2) * PAIRS_PER_W
    base_off = b * G3

    pltpu.sync_copy(pts_hbm.at[b], pts_v)
    lane = lax.iota(jnp.int32, LANES)
    c0 = jnp.zeros((LANES,), jnp.int32)
    c1 = c0 + 1
    c2 = c0 + 2
    c3 = c0 + 3

    def plane_loop(j, acc):
        pltpu.sync_copy(par_hbm.at[b, p0 + j], par_v)
        nx = par_v[0, :]
        ny = par_v[1, :]
        nz = par_v[2, :]
        dd = par_v[3, :]

        def chunk_loop(c, acc):
            cbase = c * CHUNK

            def pass1(i, carry):
                o = cbase + i * LANES
                px = pts_v[0, pl.ds(o, LANES)]
                py = pts_v[1, pl.ds(o, LANES)]
                pz = pts_v[2, pl.ds(o, LANES)]
                inner = px * nx + py * ny + pz * nz + dd
                t2 = inner + inner
                rx = px - t2 * nx
                ry = py - t2 * ny
                rz = pz - t2 * nz
                ix = _floor_f32((rx + 0.5) * float(G))
                iy = _floor_f32((ry + 0.5) * float(G))
                iz = _floor_f32((rz + 0.5) * float(G))
                ii = ix * (G * G) + iy * G + iz
                ii = jnp.clip(ii, 0, G3 - 1) + base_off
                idx_v[i // (IDXROW // LANES),
                      pl.ds((i % (IDXROW // LANES)) * LANES, LANES)] = ii
                lo = i * LANES
                rx_v[pl.ds(lo, LANES)] = rx
                ry_v[pl.ds(lo, LANES)] = ry
                rz_v[pl.ds(lo, LANES)] = rz
                return carry

            lax.fori_loop(0, CHUNK // LANES, pass1, 0)

            copies = [
                pltpu.async_copy(tab_hbm.at[idx_v.at[k]],
                                 rows_v.at[pl.ds(k * IDXROW, IDXROW)], sem)
                for k in range(NCOPY)
            ]
            for cp in copies:
                cp.wait()

            def pass2(i, acc):
                ro = i * LANES
                rid = ro + lane
                cx = plsc.load_gather(rows_v, [rid, c0])
                cy = plsc.load_gather(rows_v, [rid, c1])
                cz = plsc.load_gather(rows_v, [rid, c2])
                vx = plsc.load_gather(rows_v, [rid, c3])
                dx = rx_v[pl.ds(ro, LANES)] - cx
                dy = ry_v[pl.ds(ro, LANES)] - cy
                dz = rz_v[pl.ds(ro, LANES)] - cz
                m = 1.0 - vx
                return acc + (dx * dx + dy * dy + dz * dz) * (m * m)

            return lax.fori_loop(0, CHUNK // LANES, pass2, acc)

        return lax.fori_loop(0, N // CHUNK, chunk_loop, acc)

    acc = lax.fori_loop(0, PAIRS_PER_W, plane_loop, jnp.zeros((LANES,), jnp.float32))
    acc_v[...] = acc
    pltpu.sync_copy(acc_v, out_hbm.at[wid])


_sc_loss = functools.partial(
    pl.kernel,
    out_type=jax.ShapeDtypeStruct((NW, LANES), jnp.float32),
    mesh=plsc.VectorSubcoreMesh(core_axis_name="c", subcore_axis_name="s"),
    scratch_types=[
        pltpu.VMEM((3, N), jnp.float32),
        pltpu.VMEM((4, LANES), jnp.float32),
        pltpu.VMEM((NCOPY, IDXROW), jnp.int32),
        pltpu.VMEM((CHUNK,), jnp.float32),
        pltpu.VMEM((CHUNK,), jnp.float32),
        pltpu.VMEM((CHUNK,), jnp.float32),
        pltpu.VMEM((CHUNK, 4), jnp.float32),
        pltpu.VMEM((LANES,), jnp.float32),
        pltpu.SemaphoreType.DMA,
    ],
    compiler_params=pltpu.CompilerParams(
        needs_layout_passes=False, use_tc_tiling_on_sc=False
    ),
)(_sc_body)


def kernel(points, closest_points, voxel, planes):
    eps = 1e-12
    ns = planes[..., :3]
    ds = planes[..., 3]
    ns_norm = jnp.sqrt(jnp.sum(ns * ns, axis=2, keepdims=True))
    n_unit = ns / (ns_norm + eps)
    d_unit = ds[..., None] / (ns_norm + eps)
    params = jnp.concatenate([n_unit, d_unit], axis=-1)
    params16 = jnp.broadcast_to(params[..., None], (B, P, 4, LANES))
    pts_t = jnp.transpose(points, (0, 2, 1))
    table = jnp.concatenate(
        [closest_points.reshape(B, G3, 3), voxel.reshape(B, G3, 1)], axis=-1
    ).reshape(B * G3, 4)
    partial = _sc_loss(pts_t, params16, table)
    return jnp.sum(partial) / (B * P)


# speedup vs baseline: 1.0851x; 1.0851x over previous
import functools

import jax
import jax.numpy as jnp
from jax import lax
from jax.experimental import pallas as pl
from jax.experimental.pallas import tpu as pltpu
from jax.experimental.pallas import tpu_sc as plsc

B, P, N, G = 16, 16, 16384, 64
G3 = G * G * G
LANES = 16
NW = 32
PAIRS_PER_W = (B * P) // NW
CHUNK = 2048
IDXROW = 128
NCP = (CHUNK * 3) // IDXROW
NVX = CHUNK // IDXROW


def _floor_f32(x):
    t = x.astype(jnp.int32)
    tf = t.astype(jnp.float32)
    return jnp.where(tf > x, t - 1, t)


def _sc_body(pts_hbm, par_hbm, cp_hbm, vox_hbm, out_hbm,
             pts_v, par_v, idxc_v, idxv_v, rx_v, ry_v, rz_v,
             cpg_v, voxg_v, acc_v, sem):
    cid = lax.axis_index("c")
    sid = lax.axis_index("s")
    wid = cid * 16 + sid
    b = wid // 2
    p0 = (wid % 2) * PAIRS_PER_W

    pltpu.sync_copy(pts_hbm.at[pl.ds(b * 3 * N, 3 * N)], pts_v)
    lane = lax.iota(jnp.int32, LANES)
    lane3 = lane * 3

    def plane_loop(j, acc):
        poff = (b * P + p0 + j) * 4 * LANES
        pltpu.sync_copy(par_hbm.at[pl.ds(poff, 4 * LANES)], par_v)
        nx = par_v[pl.ds(0, LANES)]
        ny = par_v[pl.ds(LANES, LANES)]
        nz = par_v[pl.ds(2 * LANES, LANES)]
        dd = par_v[pl.ds(3 * LANES, LANES)]

        def chunk_loop(c, acc):
            cbase = c * CHUNK

            def pass1(i, carry):
                o = cbase + i * LANES
                px = pts_v[pl.ds(o, LANES)]
                py = pts_v[pl.ds(N + o, LANES)]
                pz = pts_v[pl.ds(2 * N + o, LANES)]
                inner = px * nx + py * ny + pz * nz + dd
                t2 = inner + inner
                rx = px - t2 * nx
                ry = py - t2 * ny
                rz = pz - t2 * nz
                ix = _floor_f32((rx + 0.5) * float(G))
                iy = _floor_f32((ry + 0.5) * float(G))
                iz = _floor_f32((rz + 0.5) * float(G))
                cell = jnp.clip(ix * (G * G) + iy * G + iz, 0, G3 - 1)
                icp = (cell + b * G3) * 3
                lo = i * LANES
                tpos = lo * 3 + lane3
                plsc.store_scatter(idxc_v, [tpos], icp)
                plsc.store_scatter(idxc_v, [tpos + 1], icp + 1)
                plsc.store_scatter(idxc_v, [tpos + 2], icp + 2)
                idxv_v[pl.ds(lo, LANES)] = cell + b * G3
                rx_v[pl.ds(lo, LANES)] = rx
                ry_v[pl.ds(lo, LANES)] = ry
                rz_v[pl.ds(lo, LANES)] = rz
                return carry

            lax.fori_loop(0, CHUNK // LANES, pass1, 0)

            copies = [
                pltpu.async_copy(
                    cp_hbm.at[idxc_v.at[pl.ds(k * IDXROW, IDXROW)]],
                    cpg_v.at[pl.ds(k * IDXROW, IDXROW)], sem)
                for k in range(NCP)
            ] + [
                pltpu.async_copy(
                    vox_hbm.at[idxv_v.at[pl.ds(k * IDXROW, IDXROW)]],
                    voxg_v.at[pl.ds(k * IDXROW, IDXROW)], sem)
                for k in range(NVX)
            ]
            for cp in copies:
                cp.wait()

            def pass2(i, acc):
                ro = i * LANES
                fid = ro * 3 + lane3
                cx = plsc.load_gather(cpg_v, [fid])
                cy = plsc.load_gather(cpg_v, [fid + 1])
                cz = plsc.load_gather(cpg_v, [fid + 2])
                vx = voxg_v[pl.ds(ro, LANES)]
                dx = rx_v[pl.ds(ro, LANES)] - cx
                dy = ry_v[pl.ds(ro, LANES)] - cy
                dz = rz_v[pl.ds(ro, LANES)] - cz
                m = 1.0 - vx
                return acc + (dx * dx + dy * dy + dz * dz) * (m * m)

            return lax.fori_loop(0, CHUNK // LANES, pass2, acc)

        return lax.fori_loop(0, N // CHUNK, chunk_loop, acc)

    acc = lax.fori_loop(0, PAIRS_PER_W, plane_loop, jnp.zeros((LANES,), jnp.float32))
    acc_v[...] = acc
    pltpu.sync_copy(acc_v, out_hbm.at[pl.ds(wid * LANES, LANES)])


_sc_loss = functools.partial(
    pl.kernel,
    out_type=jax.ShapeDtypeStruct((NW * LANES,), jnp.float32),
    mesh=plsc.VectorSubcoreMesh(core_axis_name="c", subcore_axis_name="s"),
    scratch_types=[
        pltpu.VMEM((3 * N,), jnp.float32),
        pltpu.VMEM((4 * LANES,), jnp.float32),
        pltpu.VMEM((CHUNK * 3,), jnp.int32),
        pltpu.VMEM((CHUNK,), jnp.int32),
        pltpu.VMEM((CHUNK,), jnp.float32),
        pltpu.VMEM((CHUNK,), jnp.float32),
        pltpu.VMEM((CHUNK,), jnp.float32),
        pltpu.VMEM((CHUNK * 3,), jnp.float32),
        pltpu.VMEM((CHUNK,), jnp.float32),
        pltpu.VMEM((LANES,), jnp.float32),
        pltpu.SemaphoreType.DMA,
    ],
    compiler_params=pltpu.CompilerParams(
        needs_layout_passes=False, use_tc_tiling_on_sc=False
    ),
)(_sc_body)


def kernel(points, closest_points, voxel, planes):
    eps = 1e-12
    ns = planes[..., :3]
    ds = planes[..., 3]
    ns_norm = jnp.sqrt(jnp.sum(ns * ns, axis=2, keepdims=True))
    n_unit = ns / (ns_norm + eps)
    d_unit = ds[..., None] / (ns_norm + eps)
    params = jnp.concatenate([n_unit, d_unit], axis=-1)
    par_1d = jnp.broadcast_to(params[..., None], (B, P, 4, LANES)).reshape(-1)
    pts_1d = jnp.transpose(points, (0, 2, 1)).reshape(-1)
    cp_1d = closest_points.reshape(-1)
    vox_1d = voxel.reshape(-1)
    partial = _sc_loss(pts_1d, par_1d, cp_1d, vox_1d)
    return jnp.sum(partial) / (B * P)


# speedup vs baseline: 3.2720x; 3.0153x over previous
import functools

import jax
import jax.numpy as jnp
from jax import lax
from jax.experimental import pallas as pl
from jax.experimental.pallas import tpu as pltpu
from jax.experimental.pallas import tpu_sc as plsc

B, P, N, G = 16, 16, 16384, 64
G3 = G * G * G
LANES = 16
NW = 32
PAIRS_PER_W = (B * P) // NW
CHUNK = 2048
IDXROW = 128
NROW = CHUNK // IDXROW
GRP = IDXROW // LANES


def _floor_f32(x):
    t = x.astype(jnp.int32)
    tf = t.astype(jnp.float32)
    return jnp.where(tf > x, t - 1, t)


def _sc_body(pts_hbm, par_hbm, cpx_hbm, cpy_hbm, cpz_hbm, vox_hbm, out_hbm,
             pts_v, par_v, idx_v, rx_v, ry_v, rz_v,
             gx_v, gy_v, gz_v, gv_v, acc_v, sem):
    cid = lax.axis_index("c")
    sid = lax.axis_index("s")
    wid = cid * 16 + sid
    b = wid // 2
    p0 = (wid % 2) * PAIRS_PER_W

    pltpu.sync_copy(pts_hbm.at[pl.ds(b * 3 * N, 3 * N)], pts_v)

    def plane_loop(j, acc):
        poff = (b * P + p0 + j) * 4 * LANES
        pltpu.sync_copy(par_hbm.at[pl.ds(poff, 4 * LANES)], par_v)
        nx = par_v[pl.ds(0, LANES)]
        ny = par_v[pl.ds(LANES, LANES)]
        nz = par_v[pl.ds(2 * LANES, LANES)]
        dd = par_v[pl.ds(3 * LANES, LANES)]

        def chunk_loop(c, acc):
            cbase = c * CHUNK

            def pass1(i, carry):
                o = cbase + i * LANES
                px = pts_v[pl.ds(o, LANES)]
                py = pts_v[pl.ds(N + o, LANES)]
                pz = pts_v[pl.ds(2 * N + o, LANES)]
                inner = px * nx + py * ny + pz * nz + dd
                t2 = inner + inner
                rx = px - t2 * nx
                ry = py - t2 * ny
                rz = pz - t2 * nz
                ix = _floor_f32((rx + 0.5) * float(G))
                iy = _floor_f32((ry + 0.5) * float(G))
                iz = _floor_f32((rz + 0.5) * float(G))
                cell = jnp.clip(ix * (G * G) + iy * G + iz, 0, G3 - 1)
                lo = i * LANES
                idx_v[i // GRP, pl.ds((i % GRP) * LANES, LANES)] = cell + b * G3
                rx_v[pl.ds(lo, LANES)] = rx
                ry_v[pl.ds(lo, LANES)] = ry
                rz_v[pl.ds(lo, LANES)] = rz
                return carry

            lax.fori_loop(0, CHUNK // LANES, pass1, 0)

            copies = []
            for tab, dst in ((cpx_hbm, gx_v), (cpy_hbm, gy_v),
                             (cpz_hbm, gz_v), (vox_hbm, gv_v)):
                copies.extend(
                    pltpu.async_copy(tab.at[idx_v.at[k]],
                                     dst.at[pl.ds(k * IDXROW, IDXROW)], sem)
                    for k in range(NROW))
            for cp in copies:
                cp.wait()

            def pass2(i, acc):
                ro = i * LANES
                dx = rx_v[pl.ds(ro, LANES)] - gx_v[pl.ds(ro, LANES)]
                dy = ry_v[pl.ds(ro, LANES)] - gy_v[pl.ds(ro, LANES)]
                dz = rz_v[pl.ds(ro, LANES)] - gz_v[pl.ds(ro, LANES)]
                m = 1.0 - gv_v[pl.ds(ro, LANES)]
                return acc + (dx * dx + dy * dy + dz * dz) * (m * m)

            return lax.fori_loop(0, CHUNK // LANES, pass2, acc)

        return lax.fori_loop(0, N // CHUNK, chunk_loop, acc)

    acc = lax.fori_loop(0, PAIRS_PER_W, plane_loop, jnp.zeros((LANES,), jnp.float32))
    acc_v[...] = acc
    pltpu.sync_copy(acc_v, out_hbm.at[pl.ds(wid * LANES, LANES)])


_sc_loss = functools.partial(
    pl.kernel,
    out_type=jax.ShapeDtypeStruct((NW * LANES,), jnp.float32),
    mesh=plsc.VectorSubcoreMesh(core_axis_name="c", subcore_axis_name="s"),
    scratch_types=[
        pltpu.VMEM((3 * N,), jnp.float32),
        pltpu.VMEM((4 * LANES,), jnp.float32),
        pltpu.VMEM((NROW, IDXROW), jnp.int32),
        pltpu.VMEM((CHUNK,), jnp.float32),
        pltpu.VMEM((CHUNK,), jnp.float32),
        pltpu.VMEM((CHUNK,), jnp.float32),
        pltpu.VMEM((CHUNK,), jnp.float32),
        pltpu.VMEM((CHUNK,), jnp.float32),
        pltpu.VMEM((CHUNK,), jnp.float32),
        pltpu.VMEM((CHUNK,), jnp.float32),
        pltpu.VMEM((LANES,), jnp.float32),
        pltpu.SemaphoreType.DMA,
    ],
    compiler_params=pltpu.CompilerParams(use_tc_tiling_on_sc=False),
)(_sc_body)


def kernel(points, closest_points, voxel, planes):
    eps = 1e-12
    ns = planes[..., :3]
    ds = planes[..., 3]
    ns_norm = jnp.sqrt(jnp.sum(ns * ns, axis=2, keepdims=True))
    n_unit = ns / (ns_norm + eps)
    d_unit = ds[..., None] / (ns_norm + eps)
    params = jnp.concatenate([n_unit, d_unit], axis=-1)
    par_1d = jnp.broadcast_to(params[..., None], (B, P, 4, LANES)).reshape(-1)
    pts_1d = jnp.transpose(points, (0, 2, 1)).reshape(-1)
    cpx = closest_points[..., 0].reshape(-1)
    cpy = closest_points[..., 1].reshape(-1)
    cpz = closest_points[..., 2].reshape(-1)
    vox_1d = voxel.reshape(-1)
    partial = _sc_loss(pts_1d, par_1d, cpx, cpy, cpz, vox_1d)
    return jnp.sum(partial) / (B * P)


# speedup vs baseline: 3.3288x; 1.0174x over previous
import functools

import jax
import jax.numpy as jnp
from jax import lax
from jax.experimental import pallas as pl
from jax.experimental.pallas import tpu as pltpu
from jax.experimental.pallas import tpu_sc as plsc

B, P, N, G = 16, 16, 16384, 64
G3 = G * G * G
LANES = 16
NW = 32
PAIRS_PER_W = (B * P) // NW
CHUNK = 2048
IDXROW = 128
NROW = CHUNK // IDXROW
GRP = IDXROW // LANES


def _floor_f32(x):
    t = x.astype(jnp.int32)
    tf = t.astype(jnp.float32)
    return jnp.where(tf > x, t - 1, t)


def _sc_body(pts_hbm, par_hbm, cpx_hbm, cpy_hbm, cpz_hbm, vox_hbm, out_hbm,
             pts_v, par_v, idx_v, rx_v, ry_v, rz_v,
             gx_v, gy_v, gz_v, gv_v, acc_v, sem):
    cid = lax.axis_index("c")
    sid = lax.axis_index("s")
    wid = cid * 16 + sid
    b = wid // 2
    p0 = (wid % 2) * PAIRS_PER_W

    pltpu.sync_copy(pts_hbm.at[pl.ds(b * 3 * N, 3 * N)], pts_v)

    def plane_loop(j, acc):
        poff = (b * P + p0 + j) * 4 * LANES
        pltpu.sync_copy(par_hbm.at[pl.ds(poff, 4 * LANES)], par_v)
        nx = par_v[pl.ds(0, LANES)]
        ny = par_v[pl.ds(LANES, LANES)]
        nz = par_v[pl.ds(2 * LANES, LANES)]
        dd = par_v[pl.ds(3 * LANES, LANES)]

        def chunk_loop(c, acc):
            cbase = c * CHUNK

            def pass1(i, carry):
                o = cbase + i * LANES
                px = pts_v[pl.ds(o, LANES)]
                py = pts_v[pl.ds(N + o, LANES)]
                pz = pts_v[pl.ds(2 * N + o, LANES)]
                inner = px * nx + py * ny + pz * nz + dd
                t2 = inner + inner
                rx = px - t2 * nx
                ry = py - t2 * ny
                rz = pz - t2 * nz
                ix = _floor_f32((rx + 0.5) * float(G))
                iy = _floor_f32((ry + 0.5) * float(G))
                iz = _floor_f32((rz + 0.5) * float(G))
                cell = jnp.clip(ix * (G * G) + iy * G + iz, 0, G3 - 1)
                lo = i * LANES
                idx_v[pl.ds(lo, LANES)] = cell + b * G3
                rx_v[pl.ds(lo, LANES)] = rx
                ry_v[pl.ds(lo, LANES)] = ry
                rz_v[pl.ds(lo, LANES)] = rz
                return carry

            lax.fori_loop(0, CHUNK // LANES, pass1, 0)

            copies = [
                pltpu.async_copy(tab.at[idx_v], dst, sem)
                for tab, dst in ((cpx_hbm, gx_v), (cpy_hbm, gy_v),
                                 (cpz_hbm, gz_v), (vox_hbm, gv_v))
            ]
            for cp in copies:
                cp.wait()

            def pass2(i, acc):
                ro = i * LANES
                dx = rx_v[pl.ds(ro, LANES)] - gx_v[pl.ds(ro, LANES)]
                dy = ry_v[pl.ds(ro, LANES)] - gy_v[pl.ds(ro, LANES)]
                dz = rz_v[pl.ds(ro, LANES)] - gz_v[pl.ds(ro, LANES)]
                m = 1.0 - gv_v[pl.ds(ro, LANES)]
                return acc + (dx * dx + dy * dy + dz * dz) * (m * m)

            return lax.fori_loop(0, CHUNK // LANES, pass2, acc)

        return lax.fori_loop(0, N // CHUNK, chunk_loop, acc)

    acc = lax.fori_loop(0, PAIRS_PER_W, plane_loop, jnp.zeros((LANES,), jnp.float32))
    acc_v[...] = acc
    pltpu.sync_copy(acc_v, out_hbm.at[pl.ds(wid * LANES, LANES)])


_sc_loss = functools.partial(
    pl.kernel,
    out_type=jax.ShapeDtypeStruct((NW * LANES,), jnp.float32),
    mesh=plsc.VectorSubcoreMesh(core_axis_name="c", subcore_axis_name="s"),
    scratch_types=[
        pltpu.VMEM((3 * N,), jnp.float32),
        pltpu.VMEM((4 * LANES,), jnp.float32),
        pltpu.VMEM((CHUNK,), jnp.int32),
        pltpu.VMEM((CHUNK,), jnp.float32),
        pltpu.VMEM((CHUNK,), jnp.float32),
        pltpu.VMEM((CHUNK,), jnp.float32),
        pltpu.VMEM((CHUNK,), jnp.float32),
        pltpu.VMEM((CHUNK,), jnp.float32),
        pltpu.VMEM((CHUNK,), jnp.float32),
        pltpu.VMEM((CHUNK,), jnp.float32),
        pltpu.VMEM((LANES,), jnp.float32),
        pltpu.SemaphoreType.DMA,
    ],
    compiler_params=pltpu.CompilerParams(use_tc_tiling_on_sc=False),
)(_sc_body)


def kernel(points, closest_points, voxel, planes):
    eps = 1e-12
    ns = planes[..., :3]
    ds = planes[..., 3]
    ns_norm = jnp.sqrt(jnp.sum(ns * ns, axis=2, keepdims=True))
    n_unit = ns / (ns_norm + eps)
    d_unit = ds[..., None] / (ns_norm + eps)
    params = jnp.concatenate([n_unit, d_unit], axis=-1)
    par_1d = jnp.broadcast_to(params[..., None], (B, P, 4, LANES)).reshape(-1)
    pts_1d = jnp.transpose(points, (0, 2, 1)).reshape(-1)
    cpx = closest_points[..., 0].reshape(-1)
    cpy = closest_points[..., 1].reshape(-1)
    cpz = closest_points[..., 2].reshape(-1)
    vox_1d = voxel.reshape(-1)
    partial = _sc_loss(pts_1d, par_1d, cpx, cpy, cpz, vox_1d)
    return jnp.sum(partial) / (B * P)


# speedup vs baseline: 3.7045x; 1.1129x over previous
import functools

import jax
import jax.numpy as jnp
from jax import lax
from jax.experimental import pallas as pl
from jax.experimental.pallas import tpu as pltpu
from jax.experimental.pallas import tpu_sc as plsc

B, P, N, G = 16, 16, 16384, 64
G3 = G * G * G
LANES = 16
NW = 32
PAIRS_PER_W = (B * P) // NW
CHUNK = 2048
NCH = N // CHUNK
NT = PAIRS_PER_W * NCH


def _floor_f32(x):
    t = x.astype(jnp.int32)
    tf = t.astype(jnp.float32)
    return jnp.where(tf > x, t - 1, t)


def _sc_body(pts_hbm, par_hbm, cpx_hbm, cpy_hbm, cpz_hbm, vox_hbm, out_hbm,
             pts_v, par_v, acc_v,
             idx0, rx0, ry0, rz0, gx0, gy0, gz0, gv0, sem0,
             idx1, rx1, ry1, rz1, gx1, gy1, gz1, gv1, sem1):
    cid = lax.axis_index("c")
    sid = lax.axis_index("s")
    wid = cid * 16 + sid
    b = wid // 2
    p0 = (wid % 2) * PAIRS_PER_W

    pltpu.sync_copy(pts_hbm.at[pl.ds(b * 3 * N, 3 * N)], pts_v)
    pltpu.sync_copy(
        par_hbm.at[pl.ds((b * P + p0) * 4 * LANES, PAIRS_PER_W * 4 * LANES)],
        par_v)

    bufs = (
        (idx0, rx0, ry0, rz0, gx0, gy0, gz0, gv0, sem0),
        (idx1, rx1, ry1, rz1, gx1, gy1, gz1, gv1, sem1),
    )

    def pass1(t, buf):
        idx_b, rx_v, ry_v, rz_v = buf[0], buf[1], buf[2], buf[3]
        j = t // NCH
        cbase = (t % NCH) * CHUNK
        po = j * 4 * LANES
        nx = par_v[pl.ds(po, LANES)]
        ny = par_v[pl.ds(po + LANES, LANES)]
        nz = par_v[pl.ds(po + 2 * LANES, LANES)]
        dd = par_v[pl.ds(po + 3 * LANES, LANES)]

        def step(i, carry):
            o = cbase + i * LANES
            px = pts_v[pl.ds(o, LANES)]
            py = pts_v[pl.ds(N + o, LANES)]
            pz = pts_v[pl.ds(2 * N + o, LANES)]
            inner = px * nx + py * ny + pz * nz + dd
            t2 = inner + inner
            rx = px - t2 * nx
            ry = py - t2 * ny
            rz = pz - t2 * nz
            ix = _floor_f32((rx + 0.5) * float(G))
            iy = _floor_f32((ry + 0.5) * float(G))
            iz = _floor_f32((rz + 0.5) * float(G))
            cell = jnp.clip(ix * (G * G) + iy * G + iz, 0, G3 - 1)
            lo = i * LANES
            idx_b[pl.ds(lo, LANES)] = cell + b * G3
            rx_v[pl.ds(lo, LANES)] = rx
            ry_v[pl.ds(lo, LANES)] = ry
            rz_v[pl.ds(lo, LANES)] = rz
            return carry

        lax.fori_loop(0, CHUNK // LANES, step, 0)

    def descs(buf):
        idx_b = buf[0]
        sem = buf[8]
        return [
            pltpu.make_async_copy(tab.at[idx_b], dst, sem)
            for tab, dst in ((cpx_hbm, buf[4]), (cpy_hbm, buf[5]),
                             (cpz_hbm, buf[6]), (vox_hbm, buf[7]))
        ]

    def start_dma(buf):
        for d in descs(buf):
            d.start()

    def wait_dma(buf):
        for d in descs(buf):
            d.wait()

    def pass2(buf, acc):
        rx_v, ry_v, rz_v = buf[1], buf[2], buf[3]
        gx_v, gy_v, gz_v, gv_v = buf[4], buf[5], buf[6], buf[7]

        def step(i, acc):
            ro = i * LANES
            dx = rx_v[pl.ds(ro, LANES)] - gx_v[pl.ds(ro, LANES)]
            dy = ry_v[pl.ds(ro, LANES)] - gy_v[pl.ds(ro, LANES)]
            dz = rz_v[pl.ds(ro, LANES)] - gz_v[pl.ds(ro, LANES)]
            m = 1.0 - gv_v[pl.ds(ro, LANES)]
            return acc + (dx * dx + dy * dy + dz * dz) * (m * m)

        return lax.fori_loop(0, CHUNK // LANES, step, acc)

    pass1(0, bufs[0])
    start_dma(bufs[0])

    def body(g, acc):
        pass1(2 * g + 1, bufs[1])
        start_dma(bufs[1])
        wait_dma(bufs[0])
        acc = pass2(bufs[0], acc)
        pass1(2 * g + 2, bufs[0])
        start_dma(bufs[0])
        wait_dma(bufs[1])
        return pass2(bufs[1], acc)

    acc = lax.fori_loop(0, NT // 2 - 1, body, jnp.zeros((LANES,), jnp.float32))
    pass1(NT - 1, bufs[1])
    start_dma(bufs[1])
    wait_dma(bufs[0])
    acc = pass2(bufs[0], acc)
    wait_dma(bufs[1])
    acc = pass2(bufs[1], acc)

    acc_v[...] = acc
    pltpu.sync_copy(acc_v, out_hbm.at[pl.ds(wid * LANES, LANES)])


def _buf_types():
    return [
        pltpu.VMEM((CHUNK,), jnp.int32),
        pltpu.VMEM((CHUNK,), jnp.float32),
        pltpu.VMEM((CHUNK,), jnp.float32),
        pltpu.VMEM((CHUNK,), jnp.float32),
        pltpu.VMEM((CHUNK,), jnp.float32),
        pltpu.VMEM((CHUNK,), jnp.float32),
        pltpu.VMEM((CHUNK,), jnp.float32),
        pltpu.VMEM((CHUNK,), jnp.float32),
        pltpu.SemaphoreType.DMA,
    ]


_sc_loss = functools.partial(
    pl.kernel,
    out_type=jax.ShapeDtypeStruct((NW * LANES,), jnp.float32),
    mesh=plsc.VectorSubcoreMesh(core_axis_name="c", subcore_axis_name="s"),
    scratch_types=[
        pltpu.VMEM((3 * N,), jnp.float32),
        pltpu.VMEM((PAIRS_PER_W * 4 * LANES,), jnp.float32),
        pltpu.VMEM((LANES,), jnp.float32),
    ] + _buf_types() + _buf_types(),
    compiler_params=pltpu.CompilerParams(use_tc_tiling_on_sc=False),
)(_sc_body)


def kernel(points, closest_points, voxel, planes):
    eps = 1e-12
    ns = planes[..., :3]
    ds = planes[..., 3]
    ns_norm = jnp.sqrt(jnp.sum(ns * ns, axis=2, keepdims=True))
    n_unit = ns / (ns_norm + eps)
    d_unit = ds[..., None] / (ns_norm + eps)
    params = jnp.concatenate([n_unit, d_unit], axis=-1)
    par_1d = jnp.broadcast_to(params[..., None], (B, P, 4, LANES)).reshape(-1)
    pts_1d = jnp.transpose(points, (0, 2, 1)).reshape(-1)
    cpx = closest_points[..., 0].reshape(-1)
    cpy = closest_points[..., 1].reshape(-1)
    cpz = closest_points[..., 2].reshape(-1)
    vox_1d = voxel.reshape(-1)
    partial = _sc_loss(pts_1d, par_1d, cpx, cpy, cpz, vox_1d)
    return jnp.sum(partial) / (B * P)
